# Initial kernel scaffold; baseline (speedup 1.0000x reference)
import jax
import jax.numpy as jnp
from jax.experimental import pallas as pl


def kernel(x, edge_index, edge_weight, batch, Ws, fc1, fc2):
    raise NotImplementedError("write your pallas kernel here")


# speedup vs baseline: 3.6903x; 3.6903x over previous
import functools

import jax
import jax.numpy as jnp
from jax import lax
from jax.experimental import pallas as pl
from jax.experimental.pallas import tpu as pltpu
from jax.experimental.pallas import tpu_sc as plsc

F32 = jnp.float32
I32 = jnp.int32

NC = 2
NS = 16
NW = NC * NS
LANES = 16
EBLK = 128
BN = 256

_MESH = functools.partial(
    plsc.VectorSubcoreMesh, core_axis_name="c", subcore_axis_name="s")


def _memset_2d(ref, rows, width, value):
    vec = jnp.full((LANES,), value, F32)

    def row(i, _):
        for v in range(width // LANES):
            ref[i, pl.ds(v * LANES, LANES)] = vec
        return 0

    lax.fori_loop(0, rows, row, 0)


def _memset_1d(ref, n, value):
    vec = jnp.full((LANES,), value, F32)

    def body(i, _):
        ref[pl.ds(i * LANES, LANES)] = vec
        return 0

    lax.fori_loop(0, n // LANES, body, 0)


def _make_agg(npad, eb, nch, ch):
    rt = npad // NS
    zbr = 64
    nvec = ch // LANES

    def body(src_hbm, dst_hbm, w_hbm, h_hbm, out_hbm,
             idx_v, w_v, rows_v, zero_v, agg_sp):
        c = lax.axis_index("c")
        s = lax.axis_index("s")
        wid = c * NS + s
        _memset_2d(zero_v, zbr, ch, 0.0)

        for ci in range(nch):
            for z in range(rt // zbr):
                pltpu.sync_copy(zero_v,
                                agg_sp.at[pl.ds(s * rt + z * zbr, zbr)])
            plsc.subcore_barrier()

            def blk(b, _, ci=ci):
                pltpu.sync_copy(src_hbm.at[wid, b], idx_v.at[0])
                pltpu.sync_copy(dst_hbm.at[wid, b], idx_v.at[1])
                pltpu.sync_copy(w_hbm.at[wid, b], w_v.at[0])
                pltpu.sync_copy(h_hbm.at[ci].at[idx_v.at[0]], rows_v)

                def grp(eg, _):
                    wv = w_v[0, pl.ds(eg * LANES, LANES)]
                    for k in range(LANES):
                        wspl = jnp.broadcast_to(wv[k], (LANES,))
                        ei = eg * LANES + k
                        for v in range(nvec):
                            rows_v[ei, pl.ds(v * LANES, LANES)] = (
                                rows_v[ei, pl.ds(v * LANES, LANES)] * wspl)
                    return 0

                lax.fori_loop(0, EBLK // LANES, grp, 0)
                pltpu.sync_copy(rows_v, agg_sp.at[idx_v.at[1]], add=True)
                return 0

            lax.fori_loop(0, eb, blk, 0)
            plsc.subcore_barrier()
            pltpu.sync_copy(agg_sp.at[pl.ds(s * rt, rt)],
                            out_hbm.at[ci, c, pl.ds(s * rt, rt)])
            plsc.subcore_barrier()

    return pl.kernel(
        body,
        out_type=jax.ShapeDtypeStruct((nch, NC, npad, ch), F32),
        mesh=_MESH(),
        scratch_types=[
            pltpu.VMEM((2, EBLK), I32),
            pltpu.VMEM((1, EBLK), F32),
            pltpu.VMEM((EBLK, ch), F32),
            pltpu.VMEM((zbr, ch), F32),
            pltpu.VMEM_SHARED((npad, ch), F32),
        ],
    )


def _make_pool(npad, gp, nchl):
    tb = npad // EBLK
    per = (tb + NW - 1) // NW
    gpt = gp // NS

    def body(h_hbm, bat_hbm, outsum_hbm,
             bat_v, rows_v, zero_v, pool_sp):
        c = lax.axis_index("c")
        s = lax.axis_index("s")
        wid = c * NS + s
        _memset_2d(zero_v, gpt, 128, 0.0)
        for ci in range(nchl):
            pltpu.sync_copy(zero_v, pool_sp.at[ci, pl.ds(s * gpt, gpt)])
        plsc.subcore_barrier()

        for t in range(per):
            bid = wid + t * NW

            @pl.when(bid < tb)
            def _(bid=bid):
                pltpu.sync_copy(bat_hbm.at[bid], bat_v.at[0])
                for ci in range(nchl):
                    pltpu.sync_copy(
                        h_hbm.at[ci].at[pl.ds(bid * EBLK, EBLK)], rows_v)
                    pltpu.sync_copy(rows_v, pool_sp.at[ci].at[bat_v.at[0]],
                                    add=True)

        plsc.subcore_barrier()
        for ci in range(nchl):
            pltpu.sync_copy(pool_sp.at[ci, pl.ds(s * gpt, gpt)],
                            outsum_hbm.at[c, ci, pl.ds(s * gpt, gpt)])

    return pl.kernel(
        body,
        out_type=jax.ShapeDtypeStruct((NC, nchl, gp, 128), F32),
        mesh=_MESH(),
        scratch_types=[
            pltpu.VMEM((1, EBLK), I32),
            pltpu.VMEM((EBLK, 128), F32),
            pltpu.VMEM((gpt, 128), F32),
            pltpu.VMEM_SHARED((nchl, gp, 128), F32),
        ],
    )


def _dinv_body(degp_ref, out_ref):
    deg = degp_ref[0, 0, :, 0] + degp_ref[0, 1, :, 0] + 1.0
    dinv = jnp.where(deg > 0, lax.rsqrt(jnp.maximum(deg, 1e-12)), 0.0)
    out_ref[...] = dinv[:, None]


def _tc_dinv(degp, npad):
    return pl.pallas_call(
        _dinv_body,
        grid=(npad // BN,),
        in_specs=[pl.BlockSpec((1, NC, BN, 128), lambda j: (0, 0, j, 0))],
        out_specs=pl.BlockSpec((BN, 1), lambda j: (j, 0)),
        out_shape=jax.ShapeDtypeStruct((npad, 1), F32),
    )(degp)


def _make_cnt_body(gp):
    def body(bat_ref, out_ref):
        r = pl.program_id(0)

        @pl.when(r == 0)
        def _():
            out_ref[...] = jnp.zeros_like(out_ref)

        bn = bat_ref.shape[1]
        seg = lax.broadcasted_iota(I32, (bn, gp), 1)
        eq = (bat_ref[0, :][:, None] == seg).astype(F32)
        out_ref[...] += jnp.sum(eq, axis=0, keepdims=True)

    return body


def _tc_cnt(batf, npad, gp):
    return pl.pallas_call(
        _make_cnt_body(gp),
        grid=(npad // BN,),
        in_specs=[pl.BlockSpec((1, BN), lambda r: (0, r))],
        out_specs=pl.BlockSpec((1, gp), lambda r: (0, 0)),
        out_shape=jax.ShapeDtypeStruct((1, gp), F32),
    )(batf)


def _mm_first_body(x_ref, w_ref, dinv_ref, out_ref):
    acc = jnp.dot(x_ref[...], w_ref[...], preferred_element_type=F32,
                  precision=lax.Precision.HIGHEST)
    out_ref[0] = dinv_ref[...] * acc


def _tc_mm_first(x, w, dinv, npad, nch, ch):
    din = w.shape[0]
    return pl.pallas_call(
        _mm_first_body,
        grid=(npad // BN, nch),
        in_specs=[
            pl.BlockSpec((BN, din), lambda r, i: (r, 0)),
            pl.BlockSpec((din, ch), lambda r, i: (0, i)),
            pl.BlockSpec((BN, 1), lambda r, i: (r, 0)),
        ],
        out_specs=pl.BlockSpec((1, BN, ch), lambda r, i: (i, r, 0)),
        out_shape=jax.ShapeDtypeStruct((nch, npad, ch), F32),
    )(x, w, dinv)


def _make_layer_body(nch_in, ch_in):
    def body(agg_ref, hpp_ref, b_ref, w_ref, dinv_ref, out_ref):
        dinv = dinv_ref[...]
        parts = []
        for ci in range(nch_in):
            a = agg_ref[ci, 0] + agg_ref[ci, 1]
            pre = dinv * (a + hpp_ref[ci]) + b_ref[0, ci * ch_in:(ci + 1) * ch_in]
            parts.append(jnp.maximum(pre, 0.0))
        hin = parts[0] if nch_in == 1 else jnp.concatenate(parts, axis=1)
        acc = jnp.dot(hin, w_ref[...], preferred_element_type=F32,
                      precision=lax.Precision.HIGHEST)
        out_ref[0] = dinv * acc

    return body


def _tc_mm_layer(agg, hpp, b, w, dinv, npad, nch_in, ch_in, nch, ch):
    din = w.shape[0]
    return pl.pallas_call(
        _make_layer_body(nch_in, ch_in),
        grid=(npad // BN, nch),
        in_specs=[
            pl.BlockSpec((nch_in, NC, BN, ch_in), lambda r, i: (0, 0, r, 0)),
            pl.BlockSpec((nch_in, BN, ch_in), lambda r, i: (0, r, 0)),
            pl.BlockSpec((1, din), lambda r, i: (0, 0)),
            pl.BlockSpec((din, ch), lambda r, i: (0, i)),
            pl.BlockSpec((BN, 1), lambda r, i: (r, 0)),
        ],
        out_specs=pl.BlockSpec((1, BN, ch), lambda r, i: (i, r, 0)),
        out_shape=jax.ShapeDtypeStruct((nch, npad, ch), F32),
    )(agg, hpp, b, w, dinv)


def _make_epi_body(nch_in, ch_in):
    def body(agg_ref, hpp_ref, b_ref, dinv_ref, out_ref):
        dinv = dinv_ref[...]
        for ci in range(nch_in):
            a = agg_ref[ci, 0] + agg_ref[ci, 1]
            pre = dinv * (a + hpp_ref[ci]) + b_ref[0, ci * ch_in:(ci + 1) * ch_in]
            out_ref[ci] = jnp.maximum(pre, 0.0)

    return body


def _tc_epilogue(agg, hpp, b, dinv, npad, nch_in, ch_in):
    dout = nch_in * ch_in
    return pl.pallas_call(
        _make_epi_body(nch_in, ch_in),
        grid=(npad // BN,),
        in_specs=[
            pl.BlockSpec((nch_in, NC, BN, ch_in), lambda r: (0, 0, r, 0)),
            pl.BlockSpec((nch_in, BN, ch_in), lambda r: (0, r, 0)),
            pl.BlockSpec((1, dout), lambda r: (0, 0)),
            pl.BlockSpec((BN, 1), lambda r: (r, 0)),
        ],
        out_specs=pl.BlockSpec((nch_in, BN, ch_in), lambda r: (0, r, 0)),
        out_shape=jax.ShapeDtypeStruct((nch_in, npad, ch_in), F32),
    )(agg, hpp, b, dinv)


def _make_head_body(g):
    def body(sums_ref, cnt_ref, w1_ref, b1_ref, w2_ref, b2_ref, out_ref):
        nchl = sums_ref.shape[1]
        sums = jnp.concatenate(
            [sums_ref[0, ci, :g, :] + sums_ref[1, ci, :g, :]
             for ci in range(nchl)], axis=1)
        cnt = cnt_ref[0, :g]
        pooled = sums / jnp.maximum(cnt, 1.0)[:, None]
        h = jnp.dot(pooled, w1_ref[...], preferred_element_type=F32,
                    precision=lax.Precision.HIGHEST) + b1_ref[...]
        h = jnp.maximum(h, 0.0)
        out_ref[...] = jnp.dot(h, w2_ref[...], preferred_element_type=F32,
                               precision=lax.Precision.HIGHEST) + b2_ref[...]

    return body


def _tc_head(sums, cnt, w1, b1, w2, b2, g):
    c = w2.shape[1]
    return pl.pallas_call(
        _make_head_body(g),
        in_specs=[pl.BlockSpec(a.shape, None) for a in (sums, cnt, w1, b1, w2, b2)],
        out_specs=pl.BlockSpec((g, c), None),
        out_shape=jax.ShapeDtypeStruct((g, c), F32),
    )(sums, cnt, w1, b1, w2, b2)


def kernel(x, edge_index, edge_weight, batch, Ws, fc1, fc2):
    n, _ = x.shape
    e = edge_index.shape[1]
    g = 64
    gp = 128

    npad = ((n + BN - 1) // BN) * BN
    while npad % (NS * 8) or (npad // NS) % 64:
        npad += BN

    xp = jnp.pad(x, ((0, npad - n), (0, 0)))
    ew = NW * EBLK
    epad = ((e + ew - 1) // ew) * ew
    eb = epad // ew
    src = jnp.pad(edge_index[0], (0, epad - e)).reshape(NW, eb, EBLK)
    dst = jnp.pad(edge_index[1], (0, epad - e)).reshape(NW, eb, EBLK)
    wgt = jnp.pad(edge_weight, (0, epad - e)).reshape(NW, eb, EBLK)
    batp = jnp.pad(batch, (0, npad - n), constant_values=g)
    batp = batp.reshape(npad // EBLK, EBLK)

    ones_feat = jnp.ones((1, npad, 128), F32)
    degp = _make_agg(npad, eb, 1, 128)(src, dst, wgt, ones_feat)
    dinv = _tc_dinv(degp, npad)
    batf = jnp.pad(batch, (0, npad - n), constant_values=g).reshape(1, npad)
    cnt = _tc_cnt(batf, npad, gp)

    ch = 128
    hpp = agg = bprev = None
    nch_in = None
    for li, (w, b) in enumerate(Ws):
        din, dout = w.shape
        din_p = ((din + ch - 1) // ch) * ch
        dout_p = ((dout + ch - 1) // ch) * ch
        nch = dout_p // ch
        wp = jnp.pad(w, ((0, din_p - din), (0, dout_p - dout)))
        bp = jnp.pad(b, (0, dout_p - dout)).reshape(1, -1)
        if li == 0:
            hpp = _tc_mm_first(xp, wp, dinv, npad, nch, ch)
        else:
            hpp = _tc_mm_layer(agg, hpp, bprev, wp, dinv, npad,
                               nch_in, ch, nch, ch)
        agg = _make_agg(npad, eb, nch, ch)(src, dst, wgt, hpp)
        nch_in, bprev = nch, bp

    h_last = _tc_epilogue(agg, hpp, bprev, dinv, npad, nch_in, ch)

    sums = _make_pool(npad, gp, nch_in)(h_last, batp)
    out = _tc_head(sums, cnt, fc1[0], fc1[1].reshape(1, -1),
                   fc2[0], fc2[1].reshape(1, -1), g)
    return out
